# Initial kernel scaffold; baseline (speedup 1.0000x reference)
import jax
import jax.numpy as jnp
from jax.experimental import pallas as pl


def kernel(seq_input, token_table, position_table):
    raise NotImplementedError("write your pallas kernel here")


# speedup vs baseline: 1.0480x; 1.0480x over previous
import functools
import math

import jax
import jax.numpy as jnp
from jax import lax
from jax.experimental import pallas as pl
from jax.experimental.pallas import tpu as pltpu
from jax.experimental.pallas import tpu_sc as plsc

_LANES = 16
_PACK = 4


@functools.lru_cache(maxsize=None)
def _make_sc_embed(nw, n_chunks, clen, seq_len, n_super, d, scale):
    mesh = plsc.VectorSubcoreMesh(core_axis_name="c", subcore_axis_name="s")
    rows_per_worker = n_chunks * clen
    n_rows = nw * rows_per_worker
    vregs_per_row = d // _LANES
    wide = _PACK * d

    @functools.partial(
        pl.kernel,
        out_type=jax.ShapeDtypeStruct((n_rows, d), jnp.float32),
        mesh=mesh,
        scratch_types=[
            pltpu.VMEM((n_chunks, clen), jnp.int32),
            pltpu.VMEM((n_chunks, clen), jnp.int32),
            pltpu.VMEM((clen, wide), jnp.float32),
            pltpu.VMEM((clen, d), jnp.float32),
            pltpu.VMEM((seq_len, d), jnp.float32),
            pltpu.SemaphoreType.DMA,
        ],
    )
    def sc_embed(sup_hbm, qoff_hbm, table_hbm, pos_hbm, out_hbm,
                 sup_v, qoff_v, rows_v, out_v, pos_v, sem):
        wid = lax.axis_index("s") * mesh.num_cores + lax.axis_index("c")
        pltpu.sync_copy(sup_hbm.at[wid], sup_v)
        pltpu.sync_copy(qoff_hbm.at[wid], qoff_v)
        pltpu.sync_copy(pos_hbm, pos_v)
        row_base = wid * rows_per_worker
        lane_iota = lax.iota(jnp.int32, _LANES)

        def chunk_body(j, carry):
            pltpu.async_copy(table_hbm.at[sup_v.at[j]], rows_v, sem).wait()
            p0 = lax.rem(j * clen, seq_len)

            def compute_body(g, carry2):
                g0 = g * _LANES
                qv = qoff_v[j, pl.ds(g0, _LANES)]
                for rr in range(_LANES):
                    row = g0 + rr
                    p_row = lax.rem(p0 + row, seq_len)
                    q = qv[rr]
                    for c in range(vregs_per_row):
                        sl = pl.ds(c * _LANES, _LANES)
                        x = rows_v[row, pl.ds(q + c * _LANES, _LANES)]
                        out_v[row, sl] = x * scale + pos_v[p_row, sl]
                return carry2

            lax.fori_loop(0, clen // _LANES, compute_body, 0, unroll=False)
            pltpu.sync_copy(
                out_v, out_hbm.at[pl.ds(row_base + j * clen, clen)])
            return carry

        lax.fori_loop(0, n_chunks, chunk_body, 0, unroll=False)

    return sc_embed


def kernel(seq_input, token_table, position_table):
    b, s = seq_input.shape
    vocab, d = token_table.shape
    scale = math.sqrt(float(d))

    info = plsc.get_sparse_core_info()
    nw = info.num_cores * info.num_subcores
    n_rows = b * s
    clen = 128
    n_chunks = n_rows // (nw * clen)
    assert n_rows == nw * clen * n_chunks and vocab % _PACK == 0

    idx = seq_input.astype(jnp.int32)
    sup = (idx // _PACK).reshape(nw, n_chunks, clen)
    qoff = ((idx % _PACK) * d).reshape(nw, n_chunks, clen)
    table_w = token_table.reshape(vocab // _PACK, _PACK * d)
    fn = _make_sc_embed(nw, n_chunks, clen, s, vocab // _PACK, d, scale)
    out = fn(sup, qoff, table_w, position_table)
    return out.reshape(b, s, d)


# speedup vs baseline: 1.4660x; 1.3989x over previous
import functools
import math

import jax
import jax.numpy as jnp
from jax import lax
from jax.experimental import pallas as pl
from jax.experimental.pallas import tpu as pltpu
from jax.experimental.pallas import tpu_sc as plsc

_LANES = 16
_PACK = 4


@functools.lru_cache(maxsize=None)
def _make_sc_embed(nw, n_chunks, clen, seq_len, n_super, d, scale):
    mesh = plsc.VectorSubcoreMesh(core_axis_name="c", subcore_axis_name="s")
    rows_per_worker = n_chunks * clen
    n_rows = nw * rows_per_worker
    vregs_per_row = d // _LANES
    wide = _PACK * d
    groups = clen // _LANES
    assert n_chunks % 2 == 0 and clen % _LANES == 0

    @functools.partial(
        pl.kernel,
        out_type=jax.ShapeDtypeStruct((n_rows, d), jnp.float32),
        mesh=mesh,
        scratch_types=[
            pltpu.VMEM((n_chunks, clen), jnp.int32),
            pltpu.VMEM((2, clen), jnp.int32),
            pltpu.VMEM((2, clen, wide), jnp.float32),
            pltpu.VMEM((2, clen, d), jnp.float32),
            pltpu.VMEM((seq_len, d), jnp.float32),
            pltpu.SemaphoreType.DMA,
            pltpu.SemaphoreType.DMA,
            pltpu.SemaphoreType.DMA,
            pltpu.SemaphoreType.DMA,
        ],
    )
    def sc_embed(idx_hbm, table_hbm, pos_hbm, out_hbm,
                 idx_v, sup_v, rows_v, out_v, pos_v, g0, g1, o0, o1):
        wid = lax.axis_index("s") * mesh.num_cores + lax.axis_index("c")
        pltpu.sync_copy(idx_hbm.at[wid], idx_v)
        pltpu.sync_copy(pos_hbm, pos_v)
        row_base = wid * rows_per_worker
        gsem = (g0, g1)
        osem = (o0, o1)

        def gather(j, b, sem):
            return pltpu.make_async_copy(
                table_hbm.at[sup_v.at[b]], rows_v.at[b], sem)

        def out_copy(j, b, sem):
            return pltpu.make_async_copy(
                out_v.at[b],
                out_hbm.at[pl.ds(row_base + j * clen, clen)], sem)

        def fill_sup(j, b):
            for g in range(groups):
                sl = pl.ds(g * _LANES, _LANES)
                sup_v[b, sl] = lax.shift_right_logical(idx_v[j, sl], 2)

        def compute(j, b):
            p0 = lax.rem(j * clen, seq_len)

            def compute_body(g, carry):
                gbase = g * _LANES
                gsl = pl.ds(gbase, _LANES)
                qv = (idx_v[j, gsl] & 3) * d
                for rr in range(_LANES):
                    row = gbase + rr
                    p_row = lax.rem(p0 + row, seq_len)
                    q = qv[rr]
                    for c in range(vregs_per_row):
                        sl = pl.ds(c * _LANES, _LANES)
                        x = rows_v[b, row, pl.ds(q + c * _LANES, _LANES)]
                        out_v[b, row, sl] = x * scale + pos_v[p_row, sl]
                return carry

            lax.fori_loop(0, groups, compute_body, 0, unroll=False)

        fill_sup(0, 0)
        gather(0, 0, g0).start()

        def phase(jj, j, b):
            nxt = j + 1

            @pl.when(nxt < n_chunks)
            def _():
                fill_sup(nxt, 1 - b)
                gather(nxt, 1 - b, gsem[1 - b]).start()

            gather(j, b, gsem[b]).wait()

            @pl.when(jj > 0)
            def _():
                out_copy(j - 2, b, osem[b]).wait()

            compute(j, b)
            out_copy(j, b, osem[b]).start()

        def loop_body(jj, carry):
            phase(jj, 2 * jj, 0)
            phase(jj, 2 * jj + 1, 1)
            return carry

        lax.fori_loop(0, n_chunks // 2, loop_body, 0, unroll=False)
        out_copy(n_chunks - 2, 0, o0).wait()
        out_copy(n_chunks - 1, 1, o1).wait()

    return sc_embed


def kernel(seq_input, token_table, position_table):
    b, s = seq_input.shape
    vocab, d = token_table.shape
    scale = math.sqrt(float(d))

    info = plsc.get_sparse_core_info()
    nw = info.num_cores * info.num_subcores
    n_rows = b * s
    clen = 128
    n_chunks = n_rows // (nw * clen)
    assert n_rows == nw * clen * n_chunks and vocab % _PACK == 0

    idx = seq_input.astype(jnp.int32).reshape(nw, n_chunks, clen)
    table_w = token_table.reshape(vocab // _PACK, _PACK * d)
    fn = _make_sc_embed(nw, n_chunks, clen, s, vocab // _PACK, d, scale)
    out = fn(idx, table_w, position_table)
    return out.reshape(b, s, d)


# speedup vs baseline: 1.5418x; 1.0517x over previous
import functools
import math

import jax
import jax.numpy as jnp
from jax import lax
from jax.experimental import pallas as pl
from jax.experimental.pallas import tpu as pltpu
from jax.experimental.pallas import tpu_sc as plsc

_LANES = 16
_PACK = 4


@functools.lru_cache(maxsize=None)
def _make_sc_embed(nw, seq_len, batch, n_super, d, scale):
    mesh = plsc.VectorSubcoreMesh(core_axis_name="c", subcore_axis_name="s")
    clen = batch // nw
    vregs_per_row = d // _LANES
    wide = _PACK * d
    groups = clen // _LANES
    assert seq_len % 2 == 0 and clen % _LANES == 0

    @functools.partial(
        pl.kernel,
        out_type=jax.ShapeDtypeStruct((seq_len, batch, d), jnp.float32),
        mesh=mesh,
        scratch_types=[
            pltpu.VMEM((seq_len, clen), jnp.int32),
            pltpu.VMEM((2, clen), jnp.int32),
            pltpu.VMEM((2, clen, wide), jnp.float32),
            pltpu.VMEM((2, clen, d), jnp.float32),
            pltpu.VMEM((seq_len, d), jnp.float32),
            pltpu.SemaphoreType.DMA,
            pltpu.SemaphoreType.DMA,
            pltpu.SemaphoreType.DMA,
            pltpu.SemaphoreType.DMA,
        ],
    )
    def sc_embed(seq_hbm, table_hbm, pos_hbm, out_hbm,
                 idx_v, sup_v, rows_v, out_v, pos_v, g0, g1, o0, o1):
        wid = lax.axis_index("s") * mesh.num_cores + lax.axis_index("c")
        col0 = wid * clen
        pltpu.sync_copy(seq_hbm.at[:, pl.ds(col0, clen)], idx_v)
        pltpu.sync_copy(pos_hbm, pos_v)
        gsem = (g0, g1)
        osem = (o0, o1)

        def gather(b, sem):
            return pltpu.make_async_copy(
                table_hbm.at[sup_v.at[b]], rows_v.at[b], sem)

        def out_copy(j, b, sem):
            return pltpu.make_async_copy(
                out_v.at[b], out_hbm.at[j, pl.ds(col0, clen)], sem)

        def fill_sup(j, b):
            for g in range(groups):
                sl = pl.ds(g * _LANES, _LANES)
                sup_v[b, sl] = lax.shift_right_logical(idx_v[j, sl], 2)

        def compute(j, b):
            pos = [pos_v[j, pl.ds(c * _LANES, _LANES)]
                   for c in range(vregs_per_row)]

            def compute_body(g, carry):
                gbase = g * _LANES
                qv = (idx_v[j, pl.ds(gbase, _LANES)] & 3) * d
                for rr in range(_LANES):
                    row = gbase + rr
                    q = qv[rr]
                    for c in range(vregs_per_row):
                        x = rows_v[b, row, pl.ds(q + c * _LANES, _LANES)]
                        out_v[b, row, pl.ds(c * _LANES, _LANES)] = (
                            x * scale + pos[c])
                return carry

            lax.fori_loop(0, groups, compute_body, 0, unroll=False)

        fill_sup(0, 0)
        gather(0, g0).start()

        def phase(jj, j, b):
            nxt = j + 1

            @pl.when(nxt < seq_len)
            def _():
                fill_sup(nxt, 1 - b)
                gather(1 - b, gsem[1 - b]).start()

            gather(b, gsem[b]).wait()

            @pl.when(jj > 0)
            def _():
                out_copy(j - 2, b, osem[b]).wait()

            compute(j, b)
            out_copy(j, b, osem[b]).start()

        def loop_body(jj, carry):
            phase(jj, 2 * jj, 0)
            phase(jj, 2 * jj + 1, 1)
            return carry

        lax.fori_loop(0, seq_len // 2, loop_body, 0, unroll=False)
        out_copy(seq_len - 2, 0, o0).wait()
        out_copy(seq_len - 1, 1, o1).wait()

    return sc_embed


def kernel(seq_input, token_table, position_table):
    b, s = seq_input.shape
    vocab, d = token_table.shape
    scale = math.sqrt(float(d))

    info = plsc.get_sparse_core_info()
    nw = info.num_cores * info.num_subcores
    assert b % (nw * _LANES) == 0 and vocab % _PACK == 0

    seq_t = jnp.swapaxes(seq_input, 0, 1).astype(jnp.int32)
    table_w = token_table.reshape(vocab // _PACK, _PACK * d)
    fn = _make_sc_embed(nw, s, b, vocab // _PACK, d, scale)
    out = fn(seq_t, table_w, position_table)
    return jnp.swapaxes(out, 0, 1)


# speedup vs baseline: 1.6514x; 1.0711x over previous
import functools
import math

import jax
import jax.numpy as jnp
from jax import lax
from jax.experimental import pallas as pl
from jax.experimental.pallas import tpu as pltpu
from jax.experimental.pallas import tpu_sc as plsc

_LANES = 16
_PACK = 4


@functools.lru_cache(maxsize=None)
def _make_sc_embed(nw, seq_len, batch, n_super, d, scale):
    mesh = plsc.VectorSubcoreMesh(core_axis_name="c", subcore_axis_name="s")
    clen = batch // nw
    vregs_per_row = d // _LANES
    wide = _PACK * d
    groups = clen // _LANES
    assert seq_len % 2 == 0 and clen % _LANES == 0

    @functools.partial(
        pl.kernel,
        out_type=jax.ShapeDtypeStruct((seq_len, batch, d), jnp.float32),
        mesh=mesh,
        scratch_types=[
            pltpu.VMEM((seq_len, clen), jnp.int32),
            pltpu.VMEM((2, clen), jnp.int32),
            pltpu.VMEM((2, clen, wide), jnp.float32),
            pltpu.VMEM((2, clen, d), jnp.float32),
            pltpu.VMEM((seq_len, d), jnp.float32),
            pltpu.SemaphoreType.DMA,
            pltpu.SemaphoreType.DMA,
            pltpu.SemaphoreType.DMA,
            pltpu.SemaphoreType.DMA,
        ],
    )
    def sc_embed(seq_hbm, table_hbm, pos_hbm, out_hbm,
                 idx_v, sup_v, rows_v, out_v, pos_v, g0, g1, o0, o1):
        wid = lax.axis_index("s") * mesh.num_cores + lax.axis_index("c")
        col0 = wid * clen
        pltpu.sync_copy(seq_hbm.at[:, pl.ds(col0, clen)], idx_v)
        pltpu.sync_copy(pos_hbm, pos_v)
        gsem = (g0, g1)
        osem = (o0, o1)

        def gather(b, sem):
            return pltpu.make_async_copy(
                table_hbm.at[sup_v.at[b]], rows_v.at[b], sem)

        def out_copy(j, b, sem):
            return pltpu.make_async_copy(
                out_v.at[b], out_hbm.at[j, pl.ds(col0, clen)], sem)

        def fill_sup(j, b):
            for g in range(groups):
                sl = pl.ds(g * _LANES, _LANES)
                sup_v[b, sl] = lax.shift_right_logical(idx_v[j, sl], 2)

        def compute(j, b):
            pos = [pos_v[j, pl.ds(c * _LANES, _LANES)]
                   for c in range(vregs_per_row)]

            @plsc.parallel_loop(0, groups, unroll=2)
            def compute_body(g):
                gbase = g * _LANES
                qv = (idx_v[j, pl.ds(gbase, _LANES)] & 3) * d
                for rr in range(_LANES):
                    row = gbase + rr
                    q = qv[rr]
                    for c in range(vregs_per_row):
                        x = rows_v[b, row, pl.ds(q + c * _LANES, _LANES)]
                        out_v[b, row, pl.ds(c * _LANES, _LANES)] = (
                            x * scale + pos[c])

        fill_sup(0, 0)
        gather(0, g0).start()

        def phase(jj, j, b):
            nxt = j + 1

            @pl.when(nxt < seq_len)
            def _():
                fill_sup(nxt, 1 - b)
                gather(1 - b, gsem[1 - b]).start()

            gather(b, gsem[b]).wait()

            @pl.when(jj > 0)
            def _():
                out_copy(j - 2, b, osem[b]).wait()

            compute(j, b)
            out_copy(j, b, osem[b]).start()

        def loop_body(jj, carry):
            phase(jj, 2 * jj, 0)
            phase(jj, 2 * jj + 1, 1)
            return carry

        lax.fori_loop(0, seq_len // 2, loop_body, 0, unroll=False)
        out_copy(seq_len - 2, 0, o0).wait()
        out_copy(seq_len - 1, 1, o1).wait()

    return sc_embed


def kernel(seq_input, token_table, position_table):
    b, s = seq_input.shape
    vocab, d = token_table.shape
    scale = math.sqrt(float(d))

    info = plsc.get_sparse_core_info()
    nw = info.num_cores * info.num_subcores
    assert b % (nw * _LANES) == 0 and vocab % _PACK == 0

    seq_t = jnp.swapaxes(seq_input, 0, 1).astype(jnp.int32)
    table_w = token_table.reshape(vocab // _PACK, _PACK * d)
    fn = _make_sc_embed(nw, s, b, vocab // _PACK, d, scale)
    out = fn(seq_t, table_w, position_table)
    return jnp.swapaxes(out, 0, 1)
